# Initial kernel scaffold; baseline (speedup 1.0000x reference)
import jax
import jax.numpy as jnp
from jax.experimental import pallas as pl


def kernel(node_latents, edge_features, senders, receivers, We1, be1, We2, be2, ge, bge, Wn1, bn1, Wn2, bn2, gn, bgn):
    raise NotImplementedError("write your pallas kernel here")


# speedup vs baseline: 2.3655x; 2.3655x over previous
import functools

import jax
import jax.numpy as jnp
from jax import lax
from jax.experimental import pallas as pl
from jax.experimental.pallas import tpu as pltpu
from jax.experimental.pallas import tpu_sc as plsc

NW = 32
NC = 2
NS = 16
C = 80


def _sc_gather(nl, senders3, receivers3, E, N, D):
    NCH = senders3.shape[1]
    EPW = NCH * C
    mesh = plsc.VectorSubcoreMesh(core_axis_name="c", subcore_axis_name="s")

    @functools.partial(
        pl.kernel,
        out_type=(jax.ShapeDtypeStruct((E, D), jnp.float32),
                  jax.ShapeDtypeStruct((E, D), jnp.float32)),
        mesh=mesh,
        scratch_types=[
            pltpu.VMEM((NCH, C), jnp.int32),
            pltpu.VMEM((NCH, C), jnp.int32),
            pltpu.VMEM((C, D), jnp.float32),
            pltpu.VMEM((C, D), jnp.float32),
            pltpu.SemaphoreType.DMA,
            pltpu.SemaphoreType.DMA,
        ],
    )
    def k(nl_hbm, s_hbm, r_hbm, gs_hbm, gr_hbm, sidx, ridx, srow, rrow,
          sem_s, sem_r):
        cid = lax.axis_index("c")
        sid = lax.axis_index("s")
        wid = sid * NC + cid
        base = wid * EPW
        pltpu.sync_copy(s_hbm.at[wid], sidx)
        pltpu.sync_copy(r_hbm.at[wid], ridx)

        def body(i, carry):
            ds = pltpu.async_copy(nl_hbm.at[sidx.at[i]], srow, sem_s)
            dr = pltpu.async_copy(nl_hbm.at[ridx.at[i]], rrow, sem_r)
            ds.wait()
            dr.wait()
            off = base + i * C
            pltpu.sync_copy(srow, gs_hbm.at[pl.ds(off, C)])
            pltpu.sync_copy(rrow, gr_hbm.at[pl.ds(off, C)])
            return carry

        lax.fori_loop(0, NCH, body, 0)

    return k(nl, senders3, receivers3)


def _sc_scatter_add(new_edge, receivers3, zeros_nd, E, N, D):
    NCH = receivers3.shape[1]
    EPW = NCH * C
    SLAB = (N // NS) // 8 * 8
    TAIL_OFF = SLAB * NS
    TAIL = N - TAIL_OFF
    mesh = plsc.VectorSubcoreMesh(core_axis_name="c", subcore_axis_name="s")

    @functools.partial(
        pl.kernel,
        out_type=jax.ShapeDtypeStruct((NC, N, D), jnp.float32),
        mesh=mesh,
        scratch_types=[
            pltpu.VMEM((NCH, C), jnp.int32),
            pltpu.VMEM((C, D), jnp.float32),
            pltpu.VMEM_SHARED((N, D), jnp.float32),
        ],
    )
    def k(ne_hbm, r_hbm, z_hbm, out_hbm, ridx, rows, aggr_sh):
        cid = lax.axis_index("c")
        sid = lax.axis_index("s")
        wid = sid * NC + cid
        base = wid * EPW
        pltpu.sync_copy(z_hbm.at[pl.ds(sid * SLAB, SLAB)],
                        aggr_sh.at[pl.ds(sid * SLAB, SLAB)])

        @pl.when(sid == 0)
        def _():
            pltpu.sync_copy(z_hbm.at[pl.ds(TAIL_OFF, TAIL)],
                            aggr_sh.at[pl.ds(TAIL_OFF, TAIL)])

        plsc.subcore_barrier()
        pltpu.sync_copy(r_hbm.at[wid], ridx)

        def body(i, carry):
            pltpu.sync_copy(ne_hbm.at[pl.ds(base + i * C, C)], rows)
            pltpu.sync_copy(rows, aggr_sh.at[ridx.at[i]], add=True)
            return carry

        lax.fori_loop(0, NCH, body, 0)
        plsc.subcore_barrier()
        pltpu.sync_copy(aggr_sh.at[pl.ds(sid * SLAB, SLAB)],
                        out_hbm.at[cid].at[pl.ds(sid * SLAB, SLAB)])

        @pl.when(sid == 0)
        def _():
            pltpu.sync_copy(aggr_sh.at[pl.ds(TAIL_OFF, TAIL)],
                            out_hbm.at[cid].at[pl.ds(TAIL_OFF, TAIL)])

    return k(new_edge, receivers3, zeros_nd)


def _edge_mlp_body(gs_ref, gr_ref, ef_ref, w1_ref, b1_ref, w2_ref, b2_ref,
                   g_ref, bg_ref, ne_ref, eo_ref):
    ef = ef_ref[...]
    x = jnp.concatenate([gs_ref[...], gr_ref[...], ef], axis=-1)
    h = jnp.dot(x, w1_ref[...], preferred_element_type=jnp.float32)
    h = jnp.maximum(h + b1_ref[...], 0.0)
    h = jnp.dot(h, w2_ref[...], preferred_element_type=jnp.float32)
    h = jnp.maximum(h + b2_ref[...], 0.0)
    mu = jnp.mean(h, axis=-1, keepdims=True)
    var = jnp.mean((h - mu) ** 2, axis=-1, keepdims=True)
    ne = (h - mu) / jnp.sqrt(var + 1e-5) * g_ref[...] + bg_ref[...]
    ne_ref[...] = ne
    eo_ref[...] = ef + ne


def _tc_edge_mlp(gs, gr, ef, We1, be1, We2, be2, ge, bge, E, D, BE=2000):
    grid = (E // BE,)
    blk = pl.BlockSpec((BE, D), lambda i: (i, 0))
    full = lambda a: pl.BlockSpec(a.shape, lambda i: tuple(0 for _ in a.shape))
    return pl.pallas_call(
        _edge_mlp_body,
        grid=grid,
        in_specs=[blk, blk, blk, full(We1), full(be1), full(We2), full(be2),
                  full(ge), full(bge)],
        out_specs=[blk, blk],
        out_shape=[jax.ShapeDtypeStruct((E, D), jnp.float32),
                   jax.ShapeDtypeStruct((E, D), jnp.float32)],
        compiler_params=pltpu.CompilerParams(
            dimension_semantics=("arbitrary",)),
    )(gs, gr, ef, We1, be1, We2, be2, ge, bge)


def _node_mlp_body(nl_ref, a0_ref, a1_ref, w1_ref, b1_ref, w2_ref, b2_ref,
                   g_ref, bg_ref, out_ref):
    nl = nl_ref[...]
    aggr = a0_ref[...] + a1_ref[...]
    x = jnp.concatenate([nl, aggr], axis=-1)
    h = jnp.dot(x, w1_ref[...], preferred_element_type=jnp.float32)
    h = jnp.maximum(h + b1_ref[...], 0.0)
    h = jnp.dot(h, w2_ref[...], preferred_element_type=jnp.float32)
    h = jnp.maximum(h + b2_ref[...], 0.0)
    mu = jnp.mean(h, axis=-1, keepdims=True)
    var = jnp.mean((h - mu) ** 2, axis=-1, keepdims=True)
    nn = (h - mu) / jnp.sqrt(var + 1e-5) * g_ref[...] + bg_ref[...]
    out_ref[...] = nn + nl


def _tc_node_mlp(nl, aggr2, Wn1, bn1, Wn2, bn2, gn, bgn, N, D, BN=2000):
    grid = (N // BN,)
    blk = pl.BlockSpec((BN, D), lambda i: (i, 0))
    full = lambda a: pl.BlockSpec(a.shape, lambda i: tuple(0 for _ in a.shape))
    return pl.pallas_call(
        _node_mlp_body,
        grid=grid,
        in_specs=[blk, blk, blk, full(Wn1), full(bn1), full(Wn2), full(bn2),
                  full(gn), full(bgn)],
        out_specs=blk,
        out_shape=jax.ShapeDtypeStruct((N, D), jnp.float32),
        compiler_params=pltpu.CompilerParams(
            dimension_semantics=("arbitrary",)),
    )(nl, aggr2[0], aggr2[1], Wn1, bn1, Wn2, bn2, gn, bgn)


def kernel(node_latents, edge_features, senders, receivers, We1, be1, We2,
           be2, ge, bge, Wn1, bn1, Wn2, bn2, gn, bgn):
    B, N, D = node_latents.shape
    E = senders.shape[0]
    EPW = E // NW
    NCH = EPW // C

    nl = node_latents.reshape(N, D)
    ef = edge_features.reshape(E, D)
    s3 = senders.astype(jnp.int32).reshape(NW, NCH, C)
    r3 = receivers.astype(jnp.int32).reshape(NW, NCH, C)

    gs, gr = _sc_gather(nl, s3, r3, E, N, D)
    new_edge, edge_out = _tc_edge_mlp(
        gs, gr, ef, We1, be1.reshape(1, D), We2, be2.reshape(1, D),
        ge.reshape(1, D), bge.reshape(1, D), E, D)
    zeros_nd = jnp.zeros((N, D), jnp.float32)
    aggr2 = _sc_scatter_add(new_edge, r3, zeros_nd, E, N, D)
    node_out = _tc_node_mlp(
        nl, aggr2, Wn1, bn1.reshape(1, D), Wn2, bn2.reshape(1, D),
        gn.reshape(1, D), bgn.reshape(1, D), N, D)
    return node_out.reshape(B, N, D), edge_out.reshape(B, E, D)


# speedup vs baseline: 2.8665x; 1.2118x over previous
import functools

import jax
import jax.numpy as jnp
from jax import lax
from jax.experimental import pallas as pl
from jax.experimental.pallas import tpu as pltpu
from jax.experimental.pallas import tpu_sc as plsc

NW = 32
NC = 2
NS = 16
C = 80


def _sc_gather(nl, senders2, receivers2, E, N, D):
    EPW = E // NW
    CG = 128
    NCH = EPW // CG
    TAIL = EPW - NCH * CG
    mesh = plsc.VectorSubcoreMesh(core_axis_name="c", subcore_axis_name="s")

    @functools.partial(
        pl.kernel,
        out_type=(jax.ShapeDtypeStruct((E, D), jnp.float32),
                  jax.ShapeDtypeStruct((E, D), jnp.float32)),
        mesh=mesh,
        scratch_types=[
            pltpu.VMEM((EPW,), jnp.int32),
            pltpu.VMEM((EPW,), jnp.int32),
            pltpu.VMEM((2, CG, D), jnp.float32),
            pltpu.VMEM((2, CG, D), jnp.float32),
            pltpu.SemaphoreType.DMA,
            pltpu.SemaphoreType.DMA,
            pltpu.SemaphoreType.DMA,
            pltpu.SemaphoreType.DMA,
            pltpu.SemaphoreType.DMA,
            pltpu.SemaphoreType.DMA,
            pltpu.SemaphoreType.DMA,
            pltpu.SemaphoreType.DMA,
        ],
    )
    def k(nl_hbm, s_hbm, r_hbm, gs_hbm, gr_hbm, sidx, ridx, srow, rrow,
          sg0, sg1, rg0, rg1, sw0, sw1, rw0, rw1):
        cid = lax.axis_index("c")
        sid = lax.axis_index("s")
        wid = sid * NC + cid
        base = wid * EPW
        pltpu.sync_copy(s_hbm.at[wid], sidx)
        pltpu.sync_copy(r_hbm.at[wid], ridx)

        def fire(i, b, gsem, rsem):
            pltpu.async_copy(nl_hbm.at[sidx.at[pl.ds(i * CG, CG)]],
                             srow.at[b], gsem)
            pltpu.async_copy(nl_hbm.at[ridx.at[pl.ds(i * CG, CG)]],
                             rrow.at[b], rsem)

        def wait_gather(i, b, gsem, rsem):
            pltpu.make_async_copy(nl_hbm.at[sidx.at[pl.ds(i * CG, CG)]],
                                  srow.at[b], gsem).wait()
            pltpu.make_async_copy(nl_hbm.at[ridx.at[pl.ds(i * CG, CG)]],
                                  rrow.at[b], rsem).wait()

        def fire_write(i, b, wsem_s, wsem_r):
            off = base + i * CG
            pltpu.async_copy(srow.at[b], gs_hbm.at[pl.ds(off, CG)], wsem_s)
            pltpu.async_copy(rrow.at[b], gr_hbm.at[pl.ds(off, CG)], wsem_r)

        def wait_write(i, b, wsem_s, wsem_r):
            off = base + i * CG
            pltpu.make_async_copy(srow.at[b], gs_hbm.at[pl.ds(off, CG)],
                                  wsem_s).wait()
            pltpu.make_async_copy(rrow.at[b], gr_hbm.at[pl.ds(off, CG)],
                                  wsem_r).wait()

        fire(0, 0, sg0, rg0)
        fire(1, 1, sg1, rg1)

        def body(j, carry):
            i0 = 2 * j
            i1 = 2 * j + 1
            wait_gather(i0, 0, sg0, rg0)
            fire_write(i0, 0, sw0, rw0)
            wait_gather(i1, 1, sg1, rg1)
            fire_write(i1, 1, sw1, rw1)
            wait_write(i0, 0, sw0, rw0)

            @pl.when(i0 + 2 < NCH)
            def _():
                fire(i0 + 2, 0, sg0, rg0)

            wait_write(i1, 1, sw1, rw1)

            @pl.when(i1 + 2 < NCH)
            def _():
                fire(i1 + 2, 1, sg1, rg1)

            return carry

        lax.fori_loop(0, NCH // 2, body, 0)

        toff = NCH * CG
        pltpu.async_copy(nl_hbm.at[sidx.at[pl.ds(toff, TAIL)]],
                         srow.at[0, pl.ds(0, TAIL)], sg0)
        pltpu.async_copy(nl_hbm.at[ridx.at[pl.ds(toff, TAIL)]],
                         rrow.at[0, pl.ds(0, TAIL)], rg0)
        pltpu.make_async_copy(nl_hbm.at[sidx.at[pl.ds(toff, TAIL)]],
                              srow.at[0, pl.ds(0, TAIL)], sg0).wait()
        pltpu.make_async_copy(nl_hbm.at[ridx.at[pl.ds(toff, TAIL)]],
                              rrow.at[0, pl.ds(0, TAIL)], rg0).wait()
        pltpu.sync_copy(srow.at[0, pl.ds(0, TAIL)],
                        gs_hbm.at[pl.ds(base + toff, TAIL)])
        pltpu.sync_copy(rrow.at[0, pl.ds(0, TAIL)],
                        gr_hbm.at[pl.ds(base + toff, TAIL)])

    return k(nl, senders2, receivers2)


def _sc_scatter_add(new_edge, receivers3, zeros_nd, E, N, D):
    NCH = receivers3.shape[1]
    EPW = NCH * C
    SLAB = (N // NS) // 8 * 8
    TAIL_OFF = SLAB * NS
    TAIL = N - TAIL_OFF
    mesh = plsc.VectorSubcoreMesh(core_axis_name="c", subcore_axis_name="s")

    @functools.partial(
        pl.kernel,
        out_type=jax.ShapeDtypeStruct((NC, N, D), jnp.float32),
        mesh=mesh,
        scratch_types=[
            pltpu.VMEM((NCH, C), jnp.int32),
            pltpu.VMEM((C, D), jnp.float32),
            pltpu.VMEM_SHARED((N, D), jnp.float32),
        ],
    )
    def k(ne_hbm, r_hbm, z_hbm, out_hbm, ridx, rows, aggr_sh):
        cid = lax.axis_index("c")
        sid = lax.axis_index("s")
        wid = sid * NC + cid
        base = wid * EPW
        pltpu.sync_copy(z_hbm.at[pl.ds(sid * SLAB, SLAB)],
                        aggr_sh.at[pl.ds(sid * SLAB, SLAB)])

        @pl.when(sid == 0)
        def _():
            pltpu.sync_copy(z_hbm.at[pl.ds(TAIL_OFF, TAIL)],
                            aggr_sh.at[pl.ds(TAIL_OFF, TAIL)])

        plsc.subcore_barrier()
        pltpu.sync_copy(r_hbm.at[wid], ridx)

        def body(i, carry):
            pltpu.sync_copy(ne_hbm.at[pl.ds(base + i * C, C)], rows)
            pltpu.sync_copy(rows, aggr_sh.at[ridx.at[i]], add=True)
            return carry

        lax.fori_loop(0, NCH, body, 0)
        plsc.subcore_barrier()
        pltpu.sync_copy(aggr_sh.at[pl.ds(sid * SLAB, SLAB)],
                        out_hbm.at[cid].at[pl.ds(sid * SLAB, SLAB)])

        @pl.when(sid == 0)
        def _():
            pltpu.sync_copy(aggr_sh.at[pl.ds(TAIL_OFF, TAIL)],
                            out_hbm.at[cid].at[pl.ds(TAIL_OFF, TAIL)])

    return k(new_edge, receivers3, zeros_nd)


def _edge_mlp_body(gs_ref, gr_ref, ef_ref, w1_ref, b1_ref, w2_ref, b2_ref,
                   g_ref, bg_ref, ne_ref, eo_ref):
    ef = ef_ref[...]
    x = jnp.concatenate([gs_ref[...], gr_ref[...], ef], axis=-1)
    h = jnp.dot(x, w1_ref[...], preferred_element_type=jnp.float32)
    h = jnp.maximum(h + b1_ref[...], 0.0)
    h = jnp.dot(h, w2_ref[...], preferred_element_type=jnp.float32)
    h = jnp.maximum(h + b2_ref[...], 0.0)
    mu = jnp.mean(h, axis=-1, keepdims=True)
    var = jnp.mean((h - mu) ** 2, axis=-1, keepdims=True)
    ne = (h - mu) / jnp.sqrt(var + 1e-5) * g_ref[...] + bg_ref[...]
    ne_ref[...] = ne
    eo_ref[...] = ef + ne


def _tc_edge_mlp(gs, gr, ef, We1, be1, We2, be2, ge, bge, E, D, BE=2000):
    grid = (E // BE,)
    blk = pl.BlockSpec((BE, D), lambda i: (i, 0))
    full = lambda a: pl.BlockSpec(a.shape, lambda i: tuple(0 for _ in a.shape))
    return pl.pallas_call(
        _edge_mlp_body,
        grid=grid,
        in_specs=[blk, blk, blk, full(We1), full(be1), full(We2), full(be2),
                  full(ge), full(bge)],
        out_specs=[blk, blk],
        out_shape=[jax.ShapeDtypeStruct((E, D), jnp.float32),
                   jax.ShapeDtypeStruct((E, D), jnp.float32)],
        compiler_params=pltpu.CompilerParams(
            dimension_semantics=("arbitrary",)),
    )(gs, gr, ef, We1, be1, We2, be2, ge, bge)


def _node_mlp_body(nl_ref, a0_ref, a1_ref, w1_ref, b1_ref, w2_ref, b2_ref,
                   g_ref, bg_ref, out_ref):
    nl = nl_ref[...]
    aggr = a0_ref[...] + a1_ref[...]
    x = jnp.concatenate([nl, aggr], axis=-1)
    h = jnp.dot(x, w1_ref[...], preferred_element_type=jnp.float32)
    h = jnp.maximum(h + b1_ref[...], 0.0)
    h = jnp.dot(h, w2_ref[...], preferred_element_type=jnp.float32)
    h = jnp.maximum(h + b2_ref[...], 0.0)
    mu = jnp.mean(h, axis=-1, keepdims=True)
    var = jnp.mean((h - mu) ** 2, axis=-1, keepdims=True)
    nn = (h - mu) / jnp.sqrt(var + 1e-5) * g_ref[...] + bg_ref[...]
    out_ref[...] = nn + nl


def _tc_node_mlp(nl, aggr2, Wn1, bn1, Wn2, bn2, gn, bgn, N, D, BN=2000):
    grid = (N // BN,)
    blk = pl.BlockSpec((BN, D), lambda i: (i, 0))
    full = lambda a: pl.BlockSpec(a.shape, lambda i: tuple(0 for _ in a.shape))
    return pl.pallas_call(
        _node_mlp_body,
        grid=grid,
        in_specs=[blk, blk, blk, full(Wn1), full(bn1), full(Wn2), full(bn2),
                  full(gn), full(bgn)],
        out_specs=blk,
        out_shape=jax.ShapeDtypeStruct((N, D), jnp.float32),
        compiler_params=pltpu.CompilerParams(
            dimension_semantics=("arbitrary",)),
    )(nl, aggr2[0], aggr2[1], Wn1, bn1, Wn2, bn2, gn, bgn)


def kernel(node_latents, edge_features, senders, receivers, We1, be1, We2,
           be2, ge, bge, Wn1, bn1, Wn2, bn2, gn, bgn):
    B, N, D = node_latents.shape
    E = senders.shape[0]
    EPW = E // NW
    NCH = EPW // C

    nl = node_latents.reshape(N, D)
    ef = edge_features.reshape(E, D)
    s2 = senders.astype(jnp.int32).reshape(NW, EPW)
    r2 = receivers.astype(jnp.int32).reshape(NW, EPW)
    r3 = receivers.astype(jnp.int32).reshape(NW, NCH, C)

    gs, gr = _sc_gather(nl, s2, r2, E, N, D)
    new_edge, edge_out = _tc_edge_mlp(
        gs, gr, ef, We1, be1.reshape(1, D), We2, be2.reshape(1, D),
        ge.reshape(1, D), bge.reshape(1, D), E, D)
    zeros_nd = jnp.zeros((N, D), jnp.float32)
    aggr2 = _sc_scatter_add(new_edge, r3, zeros_nd, E, N, D)
    node_out = _tc_node_mlp(
        nl, aggr2, Wn1, bn1.reshape(1, D), Wn2, bn2.reshape(1, D),
        gn.reshape(1, D), bgn.reshape(1, D), N, D)
    return node_out.reshape(B, N, D), edge_out.reshape(B, E, D)
